# baseline (speedup 1.0000x reference)
import functools

import jax
import jax.numpy as jnp
from jax.experimental import pallas as pl
from jax.experimental.pallas import tpu as pltpu

_CHARB_EPS2 = 1.0e-6


def _stencil_kernel(w_ref, x_ref, t_ref, o_ref, *, ksize, cpb, m):
    _, H, W = x_ref.shape
    Ho = H - ksize + 1
    Wo = W - ksize + 1
    KK = ksize * ksize
    S = 8
    span16 = 2 * S
    n_full = Ho // S
    tail = Ho - n_full * S

    def strip_sum(rows, off, wv, tot8, drop):
        wnd = [rows[off + ki:off + ki + S, kj:kj + Wo]
               for ki in range(ksize) for kj in range(ksize)]
        for r in range(m):
            acc = wv[r][0] * wnd[0]
            for tap in range(1, KK):
                acc = acc + wv[r][tap] * wnd[tap]
            v = jnp.sqrt(acc * acc + _CHARB_EPS2)
            if drop:
                rowid = jax.lax.broadcasted_iota(jnp.int32, v.shape, 0)
                v = jnp.where(rowid >= drop, v, 0.0)
            tot8 = tot8 + v
        return tot8

    def chan_body(c, tot8):
        wv = [[w_ref[c * m + r, tap] for tap in range(KK)]
              for r in range(m)]

        def strip_body(i, tot8):
            base = i * S
            rows = (x_ref[c, pl.ds(base, span16)]
                    - t_ref[c, pl.ds(base, span16)])
            return strip_sum(rows, 0, wv, tot8, 0)

        tot8 = jax.lax.fori_loop(0, n_full, strip_body, tot8)
        if tail:
            base = H - span16
            off = (Ho - S) - base
            rows = (x_ref[c, base:base + span16]
                    - t_ref[c, base:base + span16])
            tot8 = strip_sum(rows, off, wv, tot8, S - tail)
        return tot8

    tot8 = jax.lax.fori_loop(0, cpb, chan_body,
                             jnp.zeros((S, Wo), jnp.float32))
    o_ref[...] = jnp.sum(tot8, axis=0, keepdims=True)[None]


def kernel(x, t, weight):
    B, C, H, W = x.shape
    OC, _, K, _ = weight.shape
    m = OC // C
    Ho, Wo = H - K + 1, W - K + 1

    x3 = x.reshape(B * C, H, W).astype(jnp.float32)
    t3 = t.reshape(B * C, H, W).astype(jnp.float32)
    w2 = weight[:, 0].astype(jnp.float32).reshape(OC, K * K)

    out = pl.pallas_call(
        functools.partial(_stencil_kernel, ksize=K, cpb=C, m=m),
        grid=(B,),
        in_specs=[
            pl.BlockSpec(memory_space=pltpu.SMEM),
            pl.BlockSpec((C, H, W), lambda b: (b, 0, 0)),
            pl.BlockSpec((C, H, W), lambda b: (b, 0, 0)),
        ],
        out_specs=pl.BlockSpec((1, 1, Wo), lambda b: (b, 0, 0)),
        out_shape=jax.ShapeDtypeStruct((B, 1, Wo), jnp.float32),
        compiler_params=pltpu.CompilerParams(
            dimension_semantics=("parallel",),
        ),
    )(w2, x3, t3)

    denom = float(B * OC * Ho * Wo)
    return jnp.sum(out) / jnp.float32(denom)


# speedup vs baseline: 1.5843x; 1.5843x over previous
import functools

import jax
import jax.numpy as jnp
from jax.experimental import pallas as pl
from jax.experimental.pallas import tpu as pltpu

_CHARB_EPS2 = 1.0e-6


def _stencil_kernel(w_ref, x_ref, t_ref, o_ref, *, ksize, cpb, m):
    _, H, W = x_ref.shape
    Ho = H - ksize + 1
    Wo = W - ksize + 1
    KK = ksize * ksize
    S = 8
    span16 = 2 * S
    n_full = Ho // S
    tail = Ho - n_full * S

    def strip_sum(rows, off, wv, tot8, drop):
        wnd = [rows[off + ki:off + ki + S, kj:kj + Wo]
               for ki in range(ksize) for kj in range(ksize)]
        for r in range(m):
            acc = wv[r][0] * wnd[0]
            for tap in range(1, KK):
                acc = acc + wv[r][tap] * wnd[tap]
            v = jnp.sqrt(acc * acc + _CHARB_EPS2)
            if drop:
                rowid = jax.lax.broadcasted_iota(jnp.int32, v.shape, 0)
                v = jnp.where(rowid >= drop, v, 0.0)
            tot8 = tot8 + v
        return tot8

    def chan_body(c, tot8):
        wv = [[w_ref[c * m + r, tap] for tap in range(KK)]
              for r in range(m)]

        def strip_body(i, tot8):
            base = i * S
            rows = (x_ref[c, pl.ds(base, span16)]
                    - t_ref[c, pl.ds(base, span16)])
            return strip_sum(rows, 0, wv, tot8, 0)

        tot8 = jax.lax.fori_loop(0, n_full, strip_body, tot8, unroll=3)
        if tail:
            base = H - span16
            off = (Ho - S) - base
            rows = (x_ref[c, base:base + span16]
                    - t_ref[c, base:base + span16])
            tot8 = strip_sum(rows, off, wv, tot8, S - tail)
        return tot8

    tot8 = jax.lax.fori_loop(0, cpb, chan_body,
                             jnp.zeros((S, Wo), jnp.float32))
    o_ref[...] = jnp.sum(tot8, axis=0, keepdims=True)[None]


def kernel(x, t, weight):
    B, C, H, W = x.shape
    OC, _, K, _ = weight.shape
    m = OC // C
    Ho, Wo = H - K + 1, W - K + 1

    x3 = x.reshape(B * C, H, W).astype(jnp.float32)
    t3 = t.reshape(B * C, H, W).astype(jnp.float32)
    w2 = weight[:, 0].astype(jnp.float32).reshape(OC, K * K)

    out = pl.pallas_call(
        functools.partial(_stencil_kernel, ksize=K, cpb=C, m=m),
        grid=(B,),
        in_specs=[
            pl.BlockSpec(memory_space=pltpu.SMEM),
            pl.BlockSpec((C, H, W), lambda b: (b, 0, 0)),
            pl.BlockSpec((C, H, W), lambda b: (b, 0, 0)),
        ],
        out_specs=pl.BlockSpec((1, 1, Wo), lambda b: (b, 0, 0)),
        out_shape=jax.ShapeDtypeStruct((B, 1, Wo), jnp.float32),
        compiler_params=pltpu.CompilerParams(
            dimension_semantics=("parallel",),
        ),
    )(w2, x3, t3)

    denom = float(B * OC * Ho * Wo)
    return jnp.sum(out) / jnp.float32(denom)


# speedup vs baseline: 2.7315x; 1.7241x over previous
import functools

import jax
import jax.numpy as jnp
from jax.experimental import pallas as pl
from jax.experimental.pallas import tpu as pltpu

_CHARB_EPS2 = 1.0e-6


def _stencil_kernel(w_ref, x_ref, t_ref, o_ref, *, ksize, cpb, m):
    _, H, W = x_ref.shape
    Ho = H - ksize + 1
    Wo = W - ksize + 1
    KK = ksize * ksize
    Q = 16
    LOAD = Q + 8
    starts = list(range(0, Ho - Q, Q)) + [Ho - Q]

    def chan_body(c, tot):
        wv = [[w_ref[c * m + r, tap] for tap in range(KK)]
              for r in range(m)]
        for idx, s in enumerate(starts):
            drop = idx * Q - s
            base = (s // 8) * 8
            off = s - base
            d3 = (x_ref[c, base:base + LOAD]
                  - t_ref[c, base:base + LOAD]).astype(jnp.bfloat16)
            planes = [d3[:, kj:kj + Wo] for kj in range(ksize)]
            wnd = [planes[kj][off + ki:off + ki + Q]
                   for ki in range(ksize) for kj in range(ksize)]
            for r in range(m):
                acc = wv[r][0].astype(jnp.bfloat16) * wnd[0]
                for tap in range(1, KK):
                    acc = acc + wv[r][tap].astype(jnp.bfloat16) * wnd[tap]
                accf = acc.astype(jnp.float32)
                v = jnp.sqrt(accf * accf + _CHARB_EPS2)
                if drop:
                    rowid = jax.lax.broadcasted_iota(jnp.int32, v.shape, 0)
                    v = jnp.where(rowid >= drop, v, 0.0)
                tot = tot + v
        return tot

    tot = jax.lax.fori_loop(0, cpb, chan_body,
                            jnp.zeros((Q, Wo), jnp.float32))
    o_ref[...] = jnp.sum(tot, axis=0, keepdims=True)[None]


def kernel(x, t, weight):
    B, C, H, W = x.shape
    OC, _, K, _ = weight.shape
    m = OC // C
    Ho, Wo = H - K + 1, W - K + 1

    x3 = x.reshape(B * C, H, W).astype(jnp.float32)
    t3 = t.reshape(B * C, H, W).astype(jnp.float32)
    w2 = weight[:, 0].astype(jnp.float32).reshape(OC, K * K)

    out = pl.pallas_call(
        functools.partial(_stencil_kernel, ksize=K, cpb=C, m=m),
        grid=(B,),
        in_specs=[
            pl.BlockSpec(memory_space=pltpu.SMEM),
            pl.BlockSpec((C, H, W), lambda b: (b, 0, 0)),
            pl.BlockSpec((C, H, W), lambda b: (b, 0, 0)),
        ],
        out_specs=pl.BlockSpec((1, 1, Wo), lambda b: (b, 0, 0)),
        out_shape=jax.ShapeDtypeStruct((B, 1, Wo), jnp.float32),
        compiler_params=pltpu.CompilerParams(
            dimension_semantics=("parallel",),
        ),
    )(w2, x3, t3)

    denom = float(B * OC * Ho * Wo)
    return jnp.sum(out) / jnp.float32(denom)


# speedup vs baseline: 3.1316x; 1.1464x over previous
import functools

import jax
import jax.numpy as jnp
from jax.experimental import pallas as pl
from jax.experimental.pallas import tpu as pltpu

_CHARB_EPS2 = 1.0e-6


def _stencil_kernel(w_ref, x_ref, t_ref, o_ref, *, ksize, cpb, m):
    _, H, W = x_ref.shape
    Ho = H - ksize + 1
    Wo = W - ksize + 1
    KK = ksize * ksize
    Q = 32
    LOAD = Q + 8
    starts = list(range(0, Ho - Q, Q)) + [Ho - Q]

    def chan_body(c, tot):
        wv = [[w_ref[c * m + r, tap] for tap in range(KK)]
              for r in range(m)]
        for idx, s in enumerate(starts):
            drop = idx * Q - s
            base = (s // 8) * 8
            off = s - base
            d3 = (x_ref[c, base:base + LOAD]
                  - t_ref[c, base:base + LOAD]).astype(jnp.bfloat16)
            planes = [d3[:, kj:kj + Wo] for kj in range(ksize)]
            wnd = [planes[kj][off + ki:off + ki + Q]
                   for ki in range(ksize) for kj in range(ksize)]
            for r in range(m):
                acc = wv[r][0] * wnd[0]
                for tap in range(1, KK):
                    acc = acc + wv[r][tap] * wnd[tap]
                accf = acc.astype(jnp.float32)
                v = jnp.sqrt(accf * accf + _CHARB_EPS2)
                if drop:
                    rowid = jax.lax.broadcasted_iota(jnp.int32, v.shape, 0)
                    v = jnp.where(rowid >= drop, v, 0.0)
                tot = tot + v
        return tot

    tot = jax.lax.fori_loop(0, cpb, chan_body,
                            jnp.zeros((Q, Wo), jnp.float32))
    o_ref[...] = jnp.sum(tot, axis=0, keepdims=True)[None]


def kernel(x, t, weight):
    B, C, H, W = x.shape
    OC, _, K, _ = weight.shape
    m = OC // C
    Ho, Wo = H - K + 1, W - K + 1

    x3 = x.reshape(B * C, H, W).astype(jnp.float32)
    t3 = t.reshape(B * C, H, W).astype(jnp.float32)
    w2 = weight[:, 0].astype(jnp.bfloat16).reshape(OC, K * K)

    out = pl.pallas_call(
        functools.partial(_stencil_kernel, ksize=K, cpb=C, m=m),
        grid=(B,),
        in_specs=[
            pl.BlockSpec(memory_space=pltpu.SMEM),
            pl.BlockSpec((C, H, W), lambda b: (b, 0, 0)),
            pl.BlockSpec((C, H, W), lambda b: (b, 0, 0)),
        ],
        out_specs=pl.BlockSpec((1, 1, Wo), lambda b: (b, 0, 0)),
        out_shape=jax.ShapeDtypeStruct((B, 1, Wo), jnp.float32),
        compiler_params=pltpu.CompilerParams(
            dimension_semantics=("parallel",),
        ),
    )(w2, x3, t3)

    denom = float(B * OC * Ho * Wo)
    return jnp.sum(out) / jnp.float32(denom)
